# Initial kernel scaffold; baseline (speedup 1.0000x reference)
import jax
import jax.numpy as jnp
from jax.experimental import pallas as pl


def kernel(dim, pos_t, pos_y, pos_x, max_t, max_y, max_x):
    raise NotImplementedError("write your pallas kernel here")


# speedup vs baseline: 2.1224x; 2.1224x over previous
import functools

import numpy as np
import jax
import jax.numpy as jnp
from jax import lax
from jax.experimental import pallas as pl
from jax.experimental.pallas import tpu as pltpu
from jax.experimental.pallas import tpu_sc as plsc

_NC, _NS = 2, 16
_NW = _NC * _NS
_CHUNK = 128

_BASE = 10000.0


def _cos_sin_tables(D, seq_end):
    inv_freq = 1.0 / (_BASE ** (np.arange(0, D, 2, dtype=np.float64) / D))
    t = np.arange(seq_end, dtype=np.float64)
    freqs = np.outer(t, inv_freq)
    freqs = np.concatenate((freqs, freqs), axis=-1)
    return (np.cos(freqs).astype(np.float32), np.sin(freqs).astype(np.float32))


_CT, _ST = _cos_sin_tables(16, 8)
_C64, _S64 = _cos_sin_tables(24, 64)


def _make_gather(N):
    assert N % (_NW * _CHUNK) == 0
    per_w = N // _NW
    n_chunks = per_w // _CHUNK
    mesh = plsc.VectorSubcoreMesh(core_axis_name="c", subcore_axis_name="s")
    f32 = jnp.float32

    @functools.partial(
        pl.kernel,
        mesh=mesh,
        compiler_params=pltpu.CompilerParams(use_tc_tiling_on_sc=False),
        out_type=[
            jax.ShapeDtypeStruct((N, 16), f32),
            jax.ShapeDtypeStruct((N, 16), f32),
            jax.ShapeDtypeStruct((N, 24), f32),
            jax.ShapeDtypeStruct((N, 24), f32),
            jax.ShapeDtypeStruct((N, 24), f32),
            jax.ShapeDtypeStruct((N, 24), f32),
        ],
        scratch_types=[
            pltpu.VMEM((_CHUNK,), jnp.int32),
            pltpu.VMEM((_CHUNK,), jnp.int32),
            pltpu.VMEM((_CHUNK,), jnp.int32),
            pltpu.VMEM((_CHUNK, 16), f32),
            pltpu.VMEM((_CHUNK, 16), f32),
            pltpu.VMEM((_CHUNK, 24), f32),
            pltpu.VMEM((_CHUNK, 24), f32),
            pltpu.VMEM((_CHUNK, 24), f32),
            pltpu.VMEM((_CHUNK, 24), f32),
            pltpu.SemaphoreType.DMA,
        ],
    )
    def gather_kernel(pt, py, px, ct, st, c64, s64,
                      o_ct, o_st, o_cy, o_sy, o_cx, o_sx,
                      it_v, iy_v, ix_v, rct, rst, rcy, rsy, rcx, rsx, sem):
        wid = lax.axis_index("s") * _NC + lax.axis_index("c")
        base = wid * per_w

        def chunk_body(i, carry):
            tok0 = base + i * _CHUNK
            pltpu.sync_copy(pt.at[pl.ds(tok0, _CHUNK)], it_v)
            pltpu.sync_copy(py.at[pl.ds(tok0, _CHUNK)], iy_v)
            pltpu.sync_copy(px.at[pl.ds(tok0, _CHUNK)], ix_v)
            cps = [
                pltpu.async_copy(ct.at[it_v], rct, sem),
                pltpu.async_copy(st.at[it_v], rst, sem),
                pltpu.async_copy(c64.at[iy_v], rcy, sem),
                pltpu.async_copy(s64.at[iy_v], rsy, sem),
                pltpu.async_copy(c64.at[ix_v], rcx, sem),
                pltpu.async_copy(s64.at[ix_v], rsx, sem),
            ]
            for cp in cps:
                cp.wait()
            pltpu.sync_copy(rct, o_ct.at[pl.ds(tok0, _CHUNK)])
            pltpu.sync_copy(rst, o_st.at[pl.ds(tok0, _CHUNK)])
            pltpu.sync_copy(rcy, o_cy.at[pl.ds(tok0, _CHUNK)])
            pltpu.sync_copy(rsy, o_sy.at[pl.ds(tok0, _CHUNK)])
            pltpu.sync_copy(rcx, o_cx.at[pl.ds(tok0, _CHUNK)])
            pltpu.sync_copy(rsx, o_sx.at[pl.ds(tok0, _CHUNK)])
            return carry

        lax.fori_loop(0, n_chunks, chunk_body, 0)

    return gather_kernel


def kernel(dim, pos_t, pos_y, pos_x, max_t, max_y, max_x):
    ntok, B = pos_t.shape
    N = ntok * B
    pt = pos_t.reshape(N).astype(jnp.int32)
    py = pos_y.reshape(N).astype(jnp.int32)
    px = pos_x.reshape(N).astype(jnp.int32)
    tabs = (jnp.asarray(_CT), jnp.asarray(_ST), jnp.asarray(_C64), jnp.asarray(_S64))
    o_ct, o_st, o_cy, o_sy, o_cx, o_sx = _make_gather(N)(pt, py, px, *tabs)
    shp16 = (ntok, B, 1, 16)
    shp24 = (ntok, B, 1, 24)
    return (o_ct.reshape(shp16), o_st.reshape(shp16),
            o_cy.reshape(shp24), o_sy.reshape(shp24),
            o_cx.reshape(shp24), o_sx.reshape(shp24))
